# baseline (speedup 1.0000x reference)
import functools

import jax
import jax.numpy as jnp
from jax import lax
from jax.experimental import pallas as pl
from jax.experimental.pallas import tpu as pltpu
from jax.experimental.pallas import tpu_sc as plsc

N_NODES = 50000
EMBED_DIM = 64
NUM_CORES = 2
NUM_SUBCORES = 16
NUM_WORKERS = NUM_CORES * NUM_SUBCORES
WRANGE = 1600
WRANGE_LAST = N_NODES - WRANGE * (NUM_WORKERS - 1)
CHUNK = 80
GRP = 16
NGRP = CHUNK // GRP

_mesh = plsc.VectorSubcoreMesh(core_axis_name="c", subcore_axis_name="s")


@functools.partial(
    pl.kernel,
    mesh=_mesh,
    out_type=jax.ShapeDtypeStruct((N_NODES, EMBED_DIM + 1), jnp.float32),
    scratch_types=[
        pltpu.VMEM((WRANGE,), jnp.int32),
        pltpu.VMEM((CHUNK, 8, EMBED_DIM), jnp.float32),
        pltpu.VMEM((CHUNK, EMBED_DIM + 1), jnp.float32),
        pltpu.VMEM((WRANGE,), jnp.float32),
        pltpu.SemaphoreType.DMA,
        pltpu.SemaphoreType.DMA,
        pltpu.SemaphoreType.DMA,
        pltpu.SemaphoreType.DMA,
        pltpu.SemaphoreType.DMA,
    ],
)
def _gather_concat(table_hbm, idx_hbm, flag_hbm, out_hbm, idx_v, blocks_v,
                   out_v, flag_v, sem_a, sem_b, sem_c, sem_d, sem_out):
    wid = lax.axis_index("s") * NUM_CORES + lax.axis_index("c")
    last_lane = lax.iota(jnp.int32, 16) == 15
    sems = (sem_a, sem_b, sem_c, sem_d)
    wbase = pl.multiple_of(wid * WRANGE, 8)
    is_last = wid == NUM_WORKERS - 1

    @pl.when(jnp.logical_not(is_last))
    def _():
        pltpu.sync_copy(idx_hbm.at[pl.ds(wbase, WRANGE)], idx_v)
        pltpu.sync_copy(flag_hbm.at[pl.ds(wbase, WRANGE)], flag_v)

    @pl.when(is_last)
    def _():
        pltpu.sync_copy(idx_hbm.at[pl.ds(wbase, WRANGE_LAST)],
                        idx_v.at[pl.ds(0, WRANGE_LAST)])
        pltpu.sync_copy(flag_hbm.at[pl.ds(wbase, WRANGE_LAST)],
                        flag_v.at[pl.ds(0, WRANGE_LAST)])

    def issue_group(off, g):
        sem = sems[g % 4]
        ivec = idx_v[pl.ds(off + g * GRP, GRP)]
        bvec = lax.bitwise_and(ivec, ~7)
        copies = []
        for t in range(GRP):
            start = pl.multiple_of(bvec[t], 8)
            copies.append(pltpu.async_copy(
                table_hbm.at[pl.ds(start, 8), :],
                blocks_v.at[g * GRP + t], sem))
        return copies

    def extract_group(off, g, copies):
        for c in copies:
            c.wait()
        ivec = idx_v[pl.ds(off + g * GRP, GRP)]
        fvec = flag_v[pl.ds(off + g * GRP, GRP)]
        svec = lax.bitwise_and(ivec, 7)
        for t in range(GRP):
            r = g * GRP + t
            for k in range(EMBED_DIM // 16):
                out_v[r, pl.ds(k * 16, 16)] = (
                    blocks_v[r, svec[t], pl.ds(k * 16, 16)])
            cur = out_v[r, pl.ds(EMBED_DIM - 15, 16)]
            out_v[r, pl.ds(EMBED_DIM - 15, 16)] = jnp.where(
                last_lane, lax.broadcast(fvec[t], (16,)), cur)

    def chunk_body(j, carry):
        off = j * CHUNK
        obase = pl.multiple_of(wbase + off, 8)
        pending = {g: issue_group(off, g) for g in range(3)}

        @pl.when(j > 0)
        def _():
            pltpu.make_async_copy(
                out_v, out_hbm.at[pl.ds(obase, CHUNK)], sem_out).wait()

        for g in range(NGRP):
            if g + 3 < NGRP:
                pending[g + 3] = issue_group(off, g + 3)
            extract_group(off, g, pending.pop(g))
        pltpu.async_copy(out_v, out_hbm.at[pl.ds(obase, CHUNK)], sem_out)
        return carry

    n_chunks = jnp.where(is_last, WRANGE_LAST // CHUNK, WRANGE // CHUNK)
    lax.fori_loop(0, n_chunks, chunk_body, 0)
    pltpu.make_async_copy(
        out_v, out_hbm.at[pl.ds(wbase, CHUNK)], sem_out).wait()


def kernel(table, indices, is_candidate):
    return _gather_concat(table, indices.astype(jnp.int32), is_candidate)


# speedup vs baseline: 1.0086x; 1.0086x over previous
import functools

import jax
import jax.numpy as jnp
from jax import lax
from jax.experimental import pallas as pl
from jax.experimental.pallas import tpu as pltpu
from jax.experimental.pallas import tpu_sc as plsc

N_NODES = 50000
EMBED_DIM = 64
NUM_CORES = 2
NUM_SUBCORES = 16
NUM_WORKERS = NUM_CORES * NUM_SUBCORES
WRANGE = 1600
WRANGE_LAST = N_NODES - WRANGE * (NUM_WORKERS - 1)
CHUNK = 80
GRP = 16
NGRP = CHUNK // GRP

_mesh = plsc.VectorSubcoreMesh(core_axis_name="c", subcore_axis_name="s")


@functools.partial(
    pl.kernel,
    mesh=_mesh,
    out_type=jax.ShapeDtypeStruct((N_NODES, EMBED_DIM + 1), jnp.float32),
    scratch_types=[
        pltpu.VMEM((WRANGE,), jnp.int32),
        pltpu.VMEM((CHUNK, 8, EMBED_DIM), jnp.float32),
        pltpu.VMEM((CHUNK, EMBED_DIM + 1), jnp.float32),
        pltpu.VMEM((WRANGE,), jnp.float32),
        pltpu.SemaphoreType.DMA,
        pltpu.SemaphoreType.DMA,
        pltpu.SemaphoreType.DMA,
        pltpu.SemaphoreType.DMA,
    ],
)
def _gather_concat(table_hbm, idx_hbm, flag_hbm, out_hbm, idx_v, blocks_v,
                   out_v, flag_v, sem_a, sem_b, sem_c, sem_out):
    wid = lax.axis_index("s") * NUM_CORES + lax.axis_index("c")
    last_lane = lax.iota(jnp.int32, 16) == 15
    sems = (sem_a, sem_b, sem_c)
    wbase = pl.multiple_of(wid * WRANGE, 8)
    is_last = wid == NUM_WORKERS - 1

    @pl.when(jnp.logical_not(is_last))
    def _():
        pltpu.sync_copy(idx_hbm.at[pl.ds(wbase, WRANGE)], idx_v)
        pltpu.sync_copy(flag_hbm.at[pl.ds(wbase, WRANGE)], flag_v)

    @pl.when(is_last)
    def _():
        pltpu.sync_copy(idx_hbm.at[pl.ds(wbase, WRANGE_LAST)],
                        idx_v.at[pl.ds(0, WRANGE_LAST)])
        pltpu.sync_copy(flag_hbm.at[pl.ds(wbase, WRANGE_LAST)],
                        flag_v.at[pl.ds(0, WRANGE_LAST)])

    def issue_group(off, g):
        sem = sems[g % 3]
        ivec = idx_v[pl.ds(off + g * GRP, GRP)]
        bvec = lax.bitwise_and(ivec, ~7)
        copies = []
        for t in range(GRP):
            start = pl.multiple_of(bvec[t], 8)
            copies.append(pltpu.async_copy(
                table_hbm.at[pl.ds(start, 8), :],
                blocks_v.at[g * GRP + t], sem))
        return copies

    def extract_group(off, g, copies):
        for c in copies:
            c.wait()
        ivec = idx_v[pl.ds(off + g * GRP, GRP)]
        fvec = flag_v[pl.ds(off + g * GRP, GRP)]
        svec = lax.bitwise_and(ivec, 7)
        for t in range(GRP):
            r = g * GRP + t
            for k in range(EMBED_DIM // 16):
                out_v[r, pl.ds(k * 16, 16)] = (
                    blocks_v[r, svec[t], pl.ds(k * 16, 16)])
            cur = out_v[r, pl.ds(EMBED_DIM - 15, 16)]
            out_v[r, pl.ds(EMBED_DIM - 15, 16)] = jnp.where(
                last_lane, lax.broadcast(fvec[t], (16,)), cur)

    def chunk_body(j, carry):
        off = j * CHUNK
        obase = pl.multiple_of(wbase + off, 8)
        pending = {0: issue_group(off, 0), 1: issue_group(off, 1)}

        @pl.when(j > 0)
        def _():
            pltpu.make_async_copy(
                out_v, out_hbm.at[pl.ds(obase, CHUNK)], sem_out).wait()

        for g in range(NGRP):
            if g + 2 < NGRP:
                pending[g + 2] = issue_group(off, g + 2)
            extract_group(off, g, pending.pop(g))
        pltpu.async_copy(out_v, out_hbm.at[pl.ds(obase, CHUNK)], sem_out)
        return carry

    n_chunks = jnp.where(is_last, WRANGE_LAST // CHUNK, WRANGE // CHUNK)
    lax.fori_loop(0, n_chunks, chunk_body, 0)
    pltpu.make_async_copy(
        out_v, out_hbm.at[pl.ds(wbase, CHUNK)], sem_out).wait()


def kernel(table, indices, is_candidate):
    return _gather_concat(table, indices.astype(jnp.int32), is_candidate)
